# baseline (speedup 1.0000x reference)
import jax
import jax.numpy as jnp
from jax import lax
from jax.experimental import pallas as pl
from jax.experimental.pallas import tpu as pltpu
from jax.experimental.pallas import tpu_sc as plsc

B = 16384
NB = B // 2
V = 1000
D = 32

NC = 2
NS = 16
NW = NC * NS
ROWS = NB // NW
HROWS = ROWS // 2
HG = HROWS // 16
XCOL = 128


def _pool_body(ids_hbm, avg_hbm, num_hbm, dur_hbm, emb_hbm, x_hbm,
               emb_v, idx_v, sv, out_v, sem, sem_o):
  c = lax.axis_index("c")
  s = lax.axis_index("s")
  wid = s * NC + c
  base = wid * ROWS

  stage = [
      pltpu.async_copy(emb_hbm, emb_v, sem),
      pltpu.async_copy(ids_hbm.at[pl.ds(base * 10, ROWS * 10)], idx_v, sem),
      pltpu.async_copy(avg_hbm.at[pl.ds(base, ROWS)], sv.at[0], sem),
      pltpu.async_copy(num_hbm.at[pl.ds(base, ROWS)], sv.at[1], sem),
      pltpu.async_copy(dur_hbm.at[pl.ds(base, ROWS)], sv.at[2], sem),
  ]
  for cp in stage:
    cp.wait()

  mask16 = jnp.full((16,), 0xFFFF, jnp.int32)

  def half(hh):
    r0 = hh * HROWS

    @plsc.parallel_loop(0, HG, unroll=1)
    def body(g):
      row0 = r0 + g * 16
      pos0 = row0 * 10
      los, his = [], []
      for m in range(10):
        w = plsc.bitcast(idx_v[pl.ds(pos0 + 32 * m, 32)], jnp.int32)
        los.append(w & mask16)
        his.append(lax.shift_right_logical(w, 16))

      def getid(p):
        v = los[p // 32] if p % 2 == 0 else his[p // 32]
        return v[(p % 32) // 2]

      for k in range(16):
        for t in range(2):
          ids = [getid(10 * k + 5 * t + j) for j in range(5)]
          for h in range(D // 16):
            cols = pl.ds(h * 16, 16)
            acc = emb_v[ids[0], cols]
            for j in range(1, 5):
              acc = acc + emb_v[ids[j], cols]
            out_v[row0 + k, pl.ds(t * D + h * 16, 16)] = acc

    for k in range(HG):
      rows = lax.iota(jnp.int32, 16) + (r0 + k * 16)
      for f in range(3):
        colv = jnp.full((16,), 2 * D + f, jnp.int32)
        plsc.store_scatter(out_v, [rows, colv], sv[f, pl.ds(r0 + k * 16, 16)])

    return pltpu.async_copy(out_v.at[pl.ds(r0, HROWS)],
                            x_hbm.at[pl.ds(base + r0, HROWS)], sem_o)

  d0 = half(0)
  d1 = half(1)
  d0.wait()
  d1.wait()


_pool = pl.kernel(
    _pool_body,
    out_type=jax.ShapeDtypeStruct((NB, XCOL), jnp.float32),
    mesh=plsc.VectorSubcoreMesh(core_axis_name="c", subcore_axis_name="s"),
    scratch_types=[
        pltpu.VMEM((V, D), jnp.float32),
        pltpu.VMEM((ROWS * 10,), jnp.int16),
        pltpu.VMEM((3, ROWS), jnp.float32),
        pltpu.VMEM((ROWS, XCOL), jnp.float32),
        pltpu.SemaphoreType.DMA,
        pltpu.SemaphoreType.DMA,
    ],
    compiler_params=pltpu.CompilerParams(
        use_tc_tiling_on_sc=False, needs_layout_passes=False),
)

BLK = 4096
GRID = NB // BLK
_PREC = lax.Precision.DEFAULT


def _mlp_body(x_ref, w1_ref, b1_ref, w2_ref, b2_ref, w3_ref, b3_ref, out_ref):
  x = x_ref[...]
  w1 = w1_ref[...]
  h1 = jnp.dot(x[:, :2 * D] * jnp.float32(0.2), w1[:2 * D],
               preferred_element_type=jnp.float32, precision=_PREC)
  h1 = h1 + jnp.dot(x[:, 2 * D:2 * D + 3], w1[2 * D:],
                    preferred_element_type=jnp.float32, precision=_PREC)
  h1 = jnp.maximum(h1 + b1_ref[...], 0.0)
  h2 = jnp.dot(h1, w2_ref[...], preferred_element_type=jnp.float32,
               precision=_PREC)
  h2 = jnp.maximum(h2 + b2_ref[...], 0.0)
  out_ref[...] = jnp.dot(h2, w3_ref[...], preferred_element_type=jnp.float32,
                         precision=_PREC) + b3_ref[0]


_mlp = pl.pallas_call(
    _mlp_body,
    grid=(GRID,),
    in_specs=[
        pl.BlockSpec((BLK, XCOL), lambda i: (i, 0)),
        pl.BlockSpec((2 * D + 3, 256), lambda i: (0, 0)),
        pl.BlockSpec((1, 256), lambda i: (0, 0)),
        pl.BlockSpec((256, 128), lambda i: (0, 0)),
        pl.BlockSpec((1, 128), lambda i: (0, 0)),
        pl.BlockSpec((128, 1), lambda i: (0, 0)),
        pl.BlockSpec((1,), lambda i: (0,)),
    ],
    out_specs=pl.BlockSpec((BLK, 1), lambda i: (i, 0)),
    out_shape=jax.ShapeDtypeStruct((NB, 1), jnp.float32),
)


def kernel(radiant_ids, dire_ids, avg_rank_tiers, num_rank_tiers, durations,
           emb, W1, b1, W2, b2, W3, b3):
  ids = jnp.concatenate([radiant_ids, dire_ids],
                        axis=1).astype(jnp.int16).reshape(B * 10)
  b1r = b1.reshape(1, 256)
  b2r = b2.reshape(1, 128)
  outs = []
  for hh in range(2):
    ii = slice(hh * NB * 10, (hh + 1) * NB * 10)
    bb = slice(hh * NB, (hh + 1) * NB)
    x = _pool(ids[ii], avg_rank_tiers[bb], num_rank_tiers[bb],
              durations[bb], emb)
    outs.append(_mlp(x, W1, b1r, W2, b2r, W3, b3))
  return jnp.concatenate(outs, axis=0).reshape(B)


# speedup vs baseline: 1.2899x; 1.2899x over previous
import jax
import jax.numpy as jnp
from jax import lax
from jax.experimental import pallas as pl
from jax.experimental.pallas import tpu as pltpu
from jax.experimental.pallas import tpu_sc as plsc

B = 16384
V = 1000
D = 32

NC = 2
NS = 16
NW = NC * NS
ROWS = B // NW
HROWS = ROWS // 2
HG = HROWS // 16
XCOL = 128


def _pool_body(ids_hbm, avg_hbm, num_hbm, dur_hbm, emb_hbm, x_hbm,
               emb_v, idx_v, sv, out_v, sem, sem_o):
  c = lax.axis_index("c")
  s = lax.axis_index("s")
  wid = s * NC + c
  base = wid * ROWS

  stage = [
      pltpu.async_copy(emb_hbm, emb_v, sem),
      pltpu.async_copy(ids_hbm.at[pl.ds(base * 10, ROWS * 10)], idx_v, sem),
      pltpu.async_copy(avg_hbm.at[pl.ds(base, ROWS)], sv.at[0], sem),
      pltpu.async_copy(num_hbm.at[pl.ds(base, ROWS)], sv.at[1], sem),
      pltpu.async_copy(dur_hbm.at[pl.ds(base, ROWS)], sv.at[2], sem),
  ]
  for cp in stage:
    cp.wait()

  mask16 = jnp.full((16,), 0xFFFF, jnp.int32)

  def half(hh):
    r0 = hh * HROWS

    @plsc.parallel_loop(0, HG, unroll=1)
    def body(g):
      row0 = r0 + g * 16
      pos0 = row0 * 10
      los, his = [], []
      for m in range(10):
        w = plsc.bitcast(idx_v[pl.ds(pos0 + 32 * m, 32)], jnp.int32)
        los.append(w & mask16)
        his.append(lax.shift_right_logical(w, 16))

      def getid(p):
        v = los[p // 32] if p % 2 == 0 else his[p // 32]
        return v[(p % 32) // 2]

      for k in range(16):
        for t in range(2):
          ids = [getid(10 * k + 5 * t + j) for j in range(5)]
          for h in range(D // 16):
            cols = pl.ds(h * 16, 16)
            acc = emb_v[ids[0], cols]
            for j in range(1, 5):
              acc = acc + emb_v[ids[j], cols]
            out_v[row0 + k, pl.ds(t * D + h * 16, 16)] = acc

    for k in range(HG):
      rows = lax.iota(jnp.int32, 16) + (r0 + k * 16)
      for f in range(3):
        colv = jnp.full((16,), 2 * D + f, jnp.int32)
        plsc.store_scatter(out_v, [rows, colv], sv[f, pl.ds(r0 + k * 16, 16)])

    return pltpu.async_copy(out_v.at[pl.ds(r0, HROWS)],
                            x_hbm.at[pl.ds(base + r0, HROWS)], sem_o)

  d0 = half(0)
  d1 = half(1)
  d0.wait()
  d1.wait()


_pool = pl.kernel(
    _pool_body,
    out_type=jax.ShapeDtypeStruct((B, XCOL), jnp.float32),
    mesh=plsc.VectorSubcoreMesh(core_axis_name="c", subcore_axis_name="s"),
    scratch_types=[
        pltpu.VMEM((V, D), jnp.float32),
        pltpu.VMEM((ROWS * 10,), jnp.int16),
        pltpu.VMEM((3, ROWS), jnp.float32),
        pltpu.VMEM((ROWS, XCOL), jnp.float32),
        pltpu.SemaphoreType.DMA,
        pltpu.SemaphoreType.DMA,
    ],
    compiler_params=pltpu.CompilerParams(
        use_tc_tiling_on_sc=False, needs_layout_passes=False),
)

BLK = 4096
GRID = B // BLK
_PREC = lax.Precision.DEFAULT


def _mlp_body(x_ref, w1_ref, b1_ref, w2_ref, b2_ref, w3_ref, b3_ref, out_ref):
  x = x_ref[...]
  w1 = w1_ref[...]
  h1 = jnp.dot(x[:, :2 * D] * jnp.float32(0.2), w1[:2 * D],
               preferred_element_type=jnp.float32, precision=_PREC)
  h1 = h1 + jnp.dot(x[:, 2 * D:2 * D + 3], w1[2 * D:],
                    preferred_element_type=jnp.float32, precision=_PREC)
  h1 = jnp.maximum(h1 + b1_ref[...], 0.0)
  h2 = jnp.dot(h1, w2_ref[...], preferred_element_type=jnp.float32,
               precision=_PREC)
  h2 = jnp.maximum(h2 + b2_ref[...], 0.0)
  out_ref[...] = jnp.dot(h2, w3_ref[...], preferred_element_type=jnp.float32,
                         precision=_PREC) + b3_ref[0]


_mlp = pl.pallas_call(
    _mlp_body,
    grid=(GRID,),
    in_specs=[
        pl.BlockSpec((BLK, XCOL), lambda i: (i, 0)),
        pl.BlockSpec((2 * D + 3, 256), lambda i: (0, 0)),
        pl.BlockSpec((1, 256), lambda i: (0, 0)),
        pl.BlockSpec((256, 128), lambda i: (0, 0)),
        pl.BlockSpec((1, 128), lambda i: (0, 0)),
        pl.BlockSpec((128, 1), lambda i: (0, 0)),
        pl.BlockSpec((1,), lambda i: (0,)),
    ],
    out_specs=pl.BlockSpec((BLK, 1), lambda i: (i, 0)),
    out_shape=jax.ShapeDtypeStruct((B, 1), jnp.float32),
)


def kernel(radiant_ids, dire_ids, avg_rank_tiers, num_rank_tiers, durations,
           emb, W1, b1, W2, b2, W3, b3):
  ids = jnp.concatenate([radiant_ids, dire_ids],
                        axis=1).astype(jnp.int16).reshape(B * 10)
  x = _pool(ids, avg_rank_tiers, num_rank_tiers, durations, emb)
  out = _mlp(x, W1, b1.reshape(1, 256), W2, b2.reshape(1, 128), W3, b3)
  return out.reshape(B)
